# baseline (speedup 1.0000x reference)
import functools

import jax
import jax.numpy as jnp
from jax import lax
from jax.experimental import pallas as pl
from jax.experimental.pallas import tpu as pltpu
from jax.experimental.pallas import tpu_sc as plsc

_H, _HD, _K = 4, 64, 1024
_TBLK = 512

_NC, _NS = 2, 16
_NW = _NC * _NS
_CH = 128


def _vq_body(x_ref, ea_ref, esq_ref, dist_ref, ind_ref, find_ref):
    iota = jax.lax.broadcasted_iota(jnp.int32, (_TBLK, _K), 1)
    inds, finds = [], []
    for h in range(_H):
        xb = x_ref[:, h * _HD:(h + 1) * _HD]
        xb_aug = jnp.concatenate([xb, xb * xb], axis=1)
        dist = jax.lax.dot_general(
            xb_aug, ea_ref[h], (((1,), (1,)), ((), ())),
            preferred_element_type=jnp.float32)
        dist = dist - esq_ref[h:h + 1, :]
        dist_ref[:, h, :] = dist

        m = jnp.max(dist, axis=1, keepdims=True)
        ind = jnp.min(jnp.where(dist == m, iota, _K), axis=1, keepdims=True)
        inds.append(ind)
        finds.append(ind + h * _K)

    ind_ref[...] = jnp.concatenate(inds, axis=1)
    find_ref[...] = jnp.concatenate(finds, axis=1)


def _gather_quant(table, find_flat, n_rows):
    bpw = n_rows // _NW
    mesh = plsc.VectorSubcoreMesh(core_axis_name="c", subcore_axis_name="s")

    @functools.partial(
        pl.kernel, mesh=mesh,
        out_type=jax.ShapeDtypeStruct((n_rows, 2 * _HD), jnp.float32),
        scratch_types=[
            pltpu.VMEM((_CH,), jnp.int32),
            pltpu.VMEM((_CH, 2 * _HD), jnp.float32),
            pltpu.SemaphoreType.DMA,
        ],
    )
    def k(table_hbm, idx_hbm, out_hbm, idx_v, rows_v, sem):
        wid = lax.axis_index("s") * _NC + lax.axis_index("c")
        for c in range(bpw // _CH):
            base = wid * bpw + c * _CH
            pltpu.sync_copy(idx_hbm.at[pl.ds(base, _CH)], idx_v)
            pltpu.async_copy(table_hbm.at[idx_v], rows_v, sem).wait()
            pltpu.sync_copy(rows_v, out_hbm.at[pl.ds(base, _CH)])

    return k(table, find_flat)


@jax.jit
def kernel(x, x_len, embed):
    B, T, D = x.shape
    BT = B * T
    xf = x.reshape(BT, D)
    n_t = BT // _TBLK
    e_sq = jnp.sum(embed * embed, axis=-1)
    e_aug = jnp.concatenate(
        [embed * 2.0, -jnp.ones_like(embed)], axis=-1)

    dist, ind, find = pl.pallas_call(
        _vq_body,
        grid=(n_t,),
        in_specs=[
            pl.BlockSpec((_TBLK, D), lambda i: (i, 0)),
            pl.BlockSpec((_H, _K, 2 * _HD), lambda i: (0, 0, 0)),
            pl.BlockSpec((_H, _K), lambda i: (0, 0)),
        ],
        out_specs=[
            pl.BlockSpec((_TBLK, _H, _K), lambda i: (i, 0, 0)),
            pl.BlockSpec((_TBLK, _H), lambda i: (i, 0)),
            pl.BlockSpec((_TBLK, _H), lambda i: (i, 0)),
        ],
        out_shape=[
            jax.ShapeDtypeStruct((BT, _H, _K), jnp.float32),
            jax.ShapeDtypeStruct((BT, _H), jnp.int32),
            jax.ShapeDtypeStruct((BT, _H), jnp.int32),
        ],
    )(xf, e_aug, e_sq)

    table = jnp.pad(embed.reshape(_H * _K, _HD), ((0, 0), (0, _HD)))
    quant = _gather_quant(table, find.reshape(BT * _H), BT * _H)[:, :_HD]

    return (quant.reshape(B, T, D),
            ind.reshape(B, T, _H),
            dist.reshape(B, T, _H, _K))


# speedup vs baseline: 1.4036x; 1.4036x over previous
import jax
import jax.numpy as jnp
from jax.experimental import pallas as pl

_H, _HD, _K = 4, 64, 1024
_TBLK = 512


def _vq_body(x_ref, ea_ref, e_ref, esq_ref, dist_ref, ind_ref, q_ref):
    iota = jax.lax.broadcasted_iota(jnp.int32, (_TBLK, _K), 1)
    inds = []
    for h in range(_H):
        xb = x_ref[:, h * _HD:(h + 1) * _HD]
        eb = e_ref[h]
        xb_aug = jnp.concatenate([xb, xb * xb], axis=1)
        dist = jax.lax.dot_general(
            xb_aug, ea_ref[h], (((1,), (1,)), ((), ())),
            preferred_element_type=jnp.float32)
        dist = dist - esq_ref[h:h + 1, :]
        dist_ref[:, h, :] = dist

        m = jnp.max(dist, axis=1, keepdims=True)
        ind = jnp.min(jnp.where(dist == m, iota, _K), axis=1, keepdims=True)
        inds.append(ind)

        onehot = (iota == ind).astype(jnp.float32)
        q = jax.lax.dot_general(
            onehot, eb, (((1,), (0,)), ((), ())),
            preferred_element_type=jnp.float32)
        q_ref[:, h * _HD:(h + 1) * _HD] = q

    ind_ref[...] = jnp.concatenate(inds, axis=1)


@jax.jit
def kernel(x, x_len, embed):
    B, T, D = x.shape
    BT = B * T
    xf = x.reshape(BT, D)
    n_t = BT // _TBLK
    e_sq = jnp.sum(embed * embed, axis=-1)
    e_aug = jnp.concatenate(
        [embed * 2.0, -jnp.ones_like(embed)], axis=-1)

    dist, ind, quant = pl.pallas_call(
        _vq_body,
        grid=(n_t,),
        in_specs=[
            pl.BlockSpec((_TBLK, D), lambda i: (i, 0)),
            pl.BlockSpec((_H, _K, 2 * _HD), lambda i: (0, 0, 0)),
            pl.BlockSpec((_H, _K, _HD), lambda i: (0, 0, 0)),
            pl.BlockSpec((_H, _K), lambda i: (0, 0)),
        ],
        out_specs=[
            pl.BlockSpec((_TBLK, _H, _K), lambda i: (i, 0, 0)),
            pl.BlockSpec((_TBLK, _H), lambda i: (i, 0)),
            pl.BlockSpec((_TBLK, D), lambda i: (i, 0)),
        ],
        out_shape=[
            jax.ShapeDtypeStruct((BT, _H, _K), jnp.float32),
            jax.ShapeDtypeStruct((BT, _H), jnp.int32),
            jax.ShapeDtypeStruct((BT, D), jnp.float32),
        ],
    )(xf, e_aug, embed, e_sq)

    return (quant.reshape(B, T, D),
            ind.reshape(B, T, _H),
            dist.reshape(B, T, _H, _K))


# speedup vs baseline: 1.4206x; 1.0121x over previous
import jax
import jax.numpy as jnp
from jax.experimental import pallas as pl
from jax.experimental.pallas import tpu as pltpu

_H, _HD, _K = 4, 64, 1024
_TBLK = 512


def _vq_body(x_ref, e_ref, dist_ref, ind_ref, q_ref, ea_s, esq_s):
    @pl.when(pl.program_id(0) == 0)
    def _init():
        for h in range(_H):
            eb0 = e_ref[h]
            ea_s[h, :, 0:_HD] = eb0 * 2.0
            ea_s[h, :, _HD:2 * _HD] = jnp.full((_K, _HD), -1.0, jnp.float32)
            esq_s[h:h + 1, :] = jnp.sum(eb0 * eb0, axis=1)[None, :]

    iota = jax.lax.broadcasted_iota(jnp.int32, (_TBLK, _K), 1)
    inds = []
    for h in range(_H):
        xb = x_ref[:, h * _HD:(h + 1) * _HD]
        eb = e_ref[h]
        xb_aug = jnp.concatenate([xb, xb * xb], axis=1)
        dist = jax.lax.dot_general(
            xb_aug, ea_s[h], (((1,), (1,)), ((), ())),
            preferred_element_type=jnp.float32)
        dist = dist - esq_s[h:h + 1, :]
        dist_ref[:, h, :] = dist

        m = jnp.max(dist, axis=1, keepdims=True)
        ind = jnp.min(jnp.where(dist == m, iota, _K), axis=1, keepdims=True)
        inds.append(ind)

        onehot = (iota == ind).astype(jnp.float32)
        q = jax.lax.dot_general(
            onehot, eb, (((1,), (0,)), ((), ())),
            preferred_element_type=jnp.float32)
        q_ref[:, h * _HD:(h + 1) * _HD] = q

    ind_ref[...] = jnp.concatenate(inds, axis=1)


@jax.jit
def kernel(x, x_len, embed):
    B, T, D = x.shape
    BT = B * T
    xf = x.reshape(BT, D)
    n_t = BT // _TBLK
    dist, ind, quant = pl.pallas_call(
        _vq_body,
        grid=(n_t,),
        in_specs=[
            pl.BlockSpec((_TBLK, D), lambda i: (i, 0)),
            pl.BlockSpec((_H, _K, _HD), lambda i: (0, 0, 0)),
        ],
        scratch_shapes=[
            pltpu.VMEM((_H, _K, 2 * _HD), jnp.float32),
            pltpu.VMEM((_H, _K), jnp.float32),
        ],
        out_specs=[
            pl.BlockSpec((_TBLK, _H, _K), lambda i: (i, 0, 0)),
            pl.BlockSpec((_TBLK, _H), lambda i: (i, 0)),
            pl.BlockSpec((_TBLK, D), lambda i: (i, 0)),
        ],
        out_shape=[
            jax.ShapeDtypeStruct((BT, _H, _K), jnp.float32),
            jax.ShapeDtypeStruct((BT, _H), jnp.int32),
            jax.ShapeDtypeStruct((BT, D), jnp.float32),
        ],
    )(xf, embed)

    return (quant.reshape(B, T, D),
            ind.reshape(B, T, _H),
            dist.reshape(B, T, _H, _K))
